# baseline (speedup 1.0000x reference)
import functools

import jax
import jax.numpy as jnp
from jax import lax
from jax.experimental import pallas as pl
from jax.experimental.pallas import tpu as pltpu
from jax.experimental.pallas import tpu_sc as plsc

B, T, D = 32, 8192, 128
TOP_K = 100
KPAD = 112
GPAD = 104
TBLK = 1024
NC, NS = 2, 16
MIN32 = -(2 ** 31)


def _score_body(x_ref, w_ref, b_ref, e_ref):
    logits = lax.dot_general(
        x_ref[...], w_ref[...],
        dimension_numbers=(((1,), (0,)), ((), ())),
        preferred_element_type=jnp.float32,
    )
    e_ref[...] = jnp.tanh(logits + b_ref[...])


def _scores(xf, W, b):
    return pl.pallas_call(
        _score_body,
        grid=(B * T // TBLK,),
        in_specs=[
            pl.BlockSpec((TBLK, D), lambda i: (i, 0)),
            pl.BlockSpec((D, 1), lambda i: (0, 0)),
            pl.BlockSpec((TBLK, 1), lambda i: (i % (T // TBLK), 0)),
        ],
        out_specs=pl.BlockSpec((TBLK, 1), lambda i: (i, 0)),
        out_shape=jax.ShapeDtypeStruct((B * T, 1), jnp.float32),
    )(xf, W, b)


def _thresh_body(e_ref, thr_ref):
    bits = lax.bitcast_convert_type(e_ref[...], jnp.int32)
    s = jnp.where(bits >= 0, bits, bits ^ jnp.int32(0x7FFFFFFF))

    def step(i, tu):
        bit = jnp.int32(31) - i
        cand = tu | (jnp.int32(1) << bit)
        cand_s = cand ^ MIN32
        cnt = jnp.sum((s >= cand_s).astype(jnp.int32), axis=1, keepdims=True)
        return jnp.where(cnt >= TOP_K, cand, tu)

    tu = lax.fori_loop(0, 32, step, jnp.zeros((B, 1), jnp.int32))
    thr_ref[...] = jnp.broadcast_to(tu ^ MIN32, (B, 16))


def _thresholds(e2):
    return pl.pallas_call(
        _thresh_body,
        out_shape=jax.ShapeDtypeStruct((B, 16), jnp.int32),
    )(e2)


def _key16(v):
    bits = lax.bitcast_convert_type(v, jnp.int32)
    return jnp.where(bits >= 0, bits, bits ^ jnp.int32(0x7FFFFFFF))


def _make_sc_topk():
    mesh = plsc.VectorSubcoreMesh(core_axis_name="c", subcore_axis_name="s")

    @functools.partial(
        pl.kernel,
        mesh=mesh,
        out_type=[
            jax.ShapeDtypeStruct((B, GPAD, D), jnp.float32),
            jax.ShapeDtypeStruct((B * KPAD,), jnp.float32),
        ],
        scratch_types=[
            pltpu.VMEM((T,), jnp.float32),
            pltpu.VMEM((16,), jnp.int32),
            pltpu.VMEM((T + 16,), jnp.int32),
            pltpu.VMEM((T + 16,), jnp.int32),
            pltpu.VMEM((GPAD,), jnp.int32),
            pltpu.VMEM((KPAD,), jnp.float32),
            pltpu.VMEM((KPAD,), jnp.float32),
            pltpu.VMEM((GPAD, D), jnp.float32),
            pltpu.SemaphoreType.DMA,
        ],
        compiler_params=pltpu.CompilerParams(needs_layout_passes=False),
    )
    def _sc_topk(e_hbm, thr_hbm, xf_hbm, feat_out, w_out,
                 e_row, thr_v, ckey, cidx, gsel, vsel, wbuf, fbuf, sem):
        b = lax.axis_index("s") * NC + lax.axis_index("c")
        pltpu.sync_copy(e_hbm.at[pl.ds(b * T, T)], e_row)
        pltpu.sync_copy(thr_hbm.at[pl.ds(b * 16, 16)], thr_v)
        t = thr_v[...]
        lanes = lax.iota(jnp.int32, 16)
        minv = jnp.full((16,), MIN32, jnp.int32)

        def compact_step(j, off):
            sk = _key16(e_row[pl.ds(j * 16, 16)])
            m = sk >= t
            pos = plsc.cumsum(jnp.where(m, 1, 0))
            dst = off + pos - 1
            plsc.store_scatter(ckey, [dst], sk, mask=m)
            plsc.store_scatter(cidx, [dst], lanes + j * 16, mask=m)
            return off + jnp.max(pos)

        c = lax.fori_loop(0, T // 16, compact_step, jnp.int32(0))
        ckey[pl.ds(c, 16)] = minv
        nv = lax.div(c + 15, jnp.int32(16))

        gsel[pl.ds(88, 16)] = jnp.zeros((16,), jnp.int32)

        def select_step(k, carry):
            def scan_vreg(j, st):
                best, bpos = st
                kv = ckey[pl.ds(j * 16, 16)]
                pos = lanes + j * 16
                upd = kv > best
                return jnp.where(upd, kv, best), jnp.where(upd, pos, bpos)

            best, bpos = lax.fori_loop(
                0, nv, scan_vreg, (minv, jnp.zeros((16,), jnp.int32)))
            m = jnp.max(best)
            p = jnp.min(jnp.where(best == jnp.full((16,), m, jnp.int32),
                                  bpos, jnp.full((16,), T, jnp.int32)))
            psplat = jnp.full((16,), p, jnp.int32)
            kv = plsc.load_gather(ckey, [psplat])
            iv = plsc.load_gather(cidx, [psplat])
            vbits = jnp.where(kv >= 0, kv, kv ^ jnp.int32(0x7FFFFFFF))
            val = lax.bitcast_convert_type(vbits, jnp.float32)
            lane0 = lanes == 0
            ksplat = jnp.full((16,), k, jnp.int32)
            plsc.store_scatter(vsel, [ksplat], val, mask=lane0)
            plsc.store_scatter(gsel, [ksplat], iv + b * T, mask=lane0)
            plsc.store_scatter(ckey, [psplat], minv, mask=lane0)
            return carry

        lax.fori_loop(0, TOP_K, select_step, jnp.int32(0))

        pltpu.async_copy(xf_hbm.at[gsel], fbuf, sem).wait()

        neg = jnp.full((16,), -1e30, jnp.float32)
        mx = neg
        for j in range(KPAD // 16):
            v = vsel[pl.ds(j * 16, 16)]
            inb = (lanes + j * 16) < TOP_K
            mx = jnp.maximum(mx, jnp.where(inb, v, neg))
        mxs = jnp.full((16,), jnp.max(mx), jnp.float32)
        acc = jnp.zeros((16,), jnp.float32)
        for j in range(KPAD // 16):
            v = vsel[pl.ds(j * 16, 16)]
            inb = (lanes + j * 16) < TOP_K
            ev = jnp.exp(jnp.where(inb, v, neg) - mxs)
            wbuf[pl.ds(j * 16, 16)] = ev
            acc = acc + ev
        ssum = jnp.full((16,), jnp.sum(acc), jnp.float32)
        for j in range(KPAD // 16):
            wbuf[pl.ds(j * 16, 16)] = wbuf[pl.ds(j * 16, 16)] / ssum

        pltpu.sync_copy(fbuf, feat_out.at[b])
        pltpu.sync_copy(wbuf, w_out.at[pl.ds(b * KPAD, KPAD)])

    return _sc_topk


def kernel(x, W, b):
    xf = x.reshape(B * T, D)
    e = _scores(xf, W, b)
    thr = _thresholds(e.reshape(B, T))
    feats_pad, wflat = _make_sc_topk()(e.reshape(B * T), thr.reshape(B * 16), xf)
    return (feats_pad[:, :TOP_K, :], wflat.reshape(B, KPAD)[:, :TOP_K, None])


# speedup vs baseline: 3.0627x; 3.0627x over previous
import functools

import jax
import jax.numpy as jnp
from jax import lax
from jax.experimental import pallas as pl
from jax.experimental.pallas import tpu as pltpu
from jax.experimental.pallas import tpu_sc as plsc

B, T, D = 32, 8192, 128
TOP_K = 100
KPAD = 112
GPAD = 104
TBLK = 1024
NC, NS = 2, 16
MIN32 = -(2 ** 31)


def _score_body(w_ref, x_ref, b_ref, e_ref):
    for i in range(T // TBLK):
        logits = lax.dot_general(
            w_ref[...], x_ref[pl.ds(i * TBLK, TBLK), :],
            dimension_numbers=(((1,), (1,)), ((), ())),
            preferred_element_type=jnp.float32,
        )
        e_ref[pl.ds(i, 1), :] = jnp.tanh(
            logits + b_ref[:, pl.ds(i * TBLK, TBLK)])


def _scores(xf, W, b):
    return pl.pallas_call(
        _score_body,
        grid=(B,),
        in_specs=[
            pl.BlockSpec((1, D), lambda i: (0, 0)),
            pl.BlockSpec((T, D), lambda i: (i, 0)),
            pl.BlockSpec((1, T), lambda i: (0, 0)),
        ],
        out_specs=pl.BlockSpec((T // TBLK, TBLK), lambda i: (i, 0)),
        out_shape=jax.ShapeDtypeStruct((B * T // TBLK, TBLK), jnp.float32),
    )(W.reshape(1, D), xf, b.reshape(1, T))


def _thresh_body(e_ref, thr_ref):
    bits = lax.bitcast_convert_type(e_ref[...], jnp.int32)
    s = jnp.where(bits >= 0, bits, bits ^ jnp.int32(0x7FFFFFFF))

    def step(i, tu):
        bit = jnp.int32(31) - i
        cand = tu | (jnp.int32(1) << bit)
        cand_s = cand ^ MIN32
        cnt = jnp.sum((s >= cand_s).astype(jnp.int32), axis=1, keepdims=True)
        return jnp.where(cnt >= TOP_K, cand, tu)

    tu = lax.fori_loop(0, 32, step, jnp.zeros((B, 1), jnp.int32))
    thr_ref[...] = jnp.broadcast_to(tu ^ MIN32, (B, 16))


def _thresholds(e2):
    return pl.pallas_call(
        _thresh_body,
        out_shape=jax.ShapeDtypeStruct((B, 16), jnp.int32),
    )(e2)


def _key16(v):
    bits = lax.bitcast_convert_type(v, jnp.int32)
    return jnp.where(bits >= 0, bits, bits ^ jnp.int32(0x7FFFFFFF))


def _make_sc_topk():
    mesh = plsc.VectorSubcoreMesh(core_axis_name="c", subcore_axis_name="s")

    @functools.partial(
        pl.kernel,
        mesh=mesh,
        out_type=[
            jax.ShapeDtypeStruct((B, GPAD, D), jnp.float32),
            jax.ShapeDtypeStruct((B * KPAD,), jnp.float32),
        ],
        scratch_types=[
            pltpu.VMEM((T,), jnp.float32),
            pltpu.VMEM((16,), jnp.int32),
            pltpu.VMEM((T + 16,), jnp.int32),
            pltpu.VMEM((T + 16,), jnp.int32),
            pltpu.VMEM((GPAD,), jnp.int32),
            pltpu.VMEM((KPAD,), jnp.float32),
            pltpu.VMEM((KPAD,), jnp.float32),
            pltpu.VMEM((GPAD, D), jnp.float32),
            pltpu.SemaphoreType.DMA,
        ],
        compiler_params=pltpu.CompilerParams(needs_layout_passes=False),
    )
    def _sc_topk(e_hbm, thr_hbm, xf_hbm, feat_out, w_out,
                 e_row, thr_v, ckey, cidx, gsel, vsel, wbuf, fbuf, sem):
        b = lax.axis_index("s") * NC + lax.axis_index("c")
        pltpu.sync_copy(e_hbm.at[pl.ds(b * T, T)], e_row)
        pltpu.sync_copy(thr_hbm.at[pl.ds(b * 16, 16)], thr_v)
        t = thr_v[...]
        lanes = lax.iota(jnp.int32, 16)
        minv = jnp.full((16,), MIN32, jnp.int32)

        def compact_step(j, off):
            sk = _key16(e_row[pl.ds(j * 16, 16)])
            m = sk >= t
            pos = plsc.cumsum(jnp.where(m, 1, 0))
            dst = off + pos - 1
            plsc.store_scatter(ckey, [dst], sk, mask=m)
            plsc.store_scatter(cidx, [dst], lanes + j * 16, mask=m)
            return off + jnp.max(pos)

        c = lax.fori_loop(0, T // 16, compact_step, jnp.int32(0))
        ckey[pl.ds(c, 16)] = minv
        nv = lax.div(c + 15, jnp.int32(16))

        gsel[pl.ds(88, 16)] = jnp.zeros((16,), jnp.int32)

        def select_step(k, carry):
            def scan_vreg(j, st):
                best, bpos = st
                kv = ckey[pl.ds(j * 16, 16)]
                pos = lanes + j * 16
                upd = kv > best
                return jnp.where(upd, kv, best), jnp.where(upd, pos, bpos)

            best, bpos = lax.fori_loop(
                0, nv, scan_vreg, (minv, jnp.zeros((16,), jnp.int32)))
            m = jnp.max(best)
            p = jnp.min(jnp.where(best == jnp.full((16,), m, jnp.int32),
                                  bpos, jnp.full((16,), T, jnp.int32)))
            psplat = jnp.full((16,), p, jnp.int32)
            kv = plsc.load_gather(ckey, [psplat])
            iv = plsc.load_gather(cidx, [psplat])
            vbits = jnp.where(kv >= 0, kv, kv ^ jnp.int32(0x7FFFFFFF))
            val = lax.bitcast_convert_type(vbits, jnp.float32)
            lane0 = lanes == 0
            ksplat = jnp.full((16,), k, jnp.int32)
            plsc.store_scatter(vsel, [ksplat], val, mask=lane0)
            plsc.store_scatter(gsel, [ksplat], iv + b * T, mask=lane0)
            plsc.store_scatter(ckey, [psplat], minv, mask=lane0)
            return carry

        lax.fori_loop(0, TOP_K, select_step, jnp.int32(0))

        pltpu.async_copy(xf_hbm.at[gsel], fbuf, sem).wait()

        neg = jnp.full((16,), -1e30, jnp.float32)
        mx = neg
        for j in range(KPAD // 16):
            v = vsel[pl.ds(j * 16, 16)]
            inb = (lanes + j * 16) < TOP_K
            mx = jnp.maximum(mx, jnp.where(inb, v, neg))
        mxs = jnp.full((16,), jnp.max(mx), jnp.float32)
        acc = jnp.zeros((16,), jnp.float32)
        for j in range(KPAD // 16):
            v = vsel[pl.ds(j * 16, 16)]
            inb = (lanes + j * 16) < TOP_K
            ev = jnp.exp(jnp.where(inb, v, neg) - mxs)
            wbuf[pl.ds(j * 16, 16)] = ev
            acc = acc + ev
        ssum = jnp.full((16,), jnp.sum(acc), jnp.float32)
        for j in range(KPAD // 16):
            wbuf[pl.ds(j * 16, 16)] = wbuf[pl.ds(j * 16, 16)] / ssum

        pltpu.sync_copy(fbuf, feat_out.at[b])
        pltpu.sync_copy(wbuf, w_out.at[pl.ds(b * KPAD, KPAD)])

    return _sc_topk


def kernel(x, W, b):
    xf = x.reshape(B * T, D)
    e = _scores(xf, W, b)
    thr = _thresholds(e.reshape(B, T))
    feats_pad, wflat = _make_sc_topk()(e.reshape(B * T), thr.reshape(B * 16), xf)
    return (feats_pad[:, :TOP_K, :], wflat.reshape(B, KPAD)[:, :TOP_K, None])
